# Initial kernel scaffold; baseline (speedup 1.0000x reference)
import jax
import jax.numpy as jnp
from jax.experimental import pallas as pl


def kernel(x, edge_index, batch, embed_W, embed_b, conv_W, conv_b, fc_W, fc_b):
    raise NotImplementedError("write your pallas kernel here")


# speedup vs baseline: 14.2772x; 14.2772x over previous
import functools

import jax
import jax.numpy as jnp
from jax import lax
from jax.experimental import pallas as pl
from jax.experimental.pallas import tpu as pltpu
from jax.experimental.pallas import tpu_sc as plsc

N = 100000
E = 1600000
IN_F = 2
H = 32
HH = 16
L_LAYERS = 9
OUT_F = 2
G = 64

NC = 2
NS = 16
LANES = 16

STREAM = 128
KB = 8
BLK = STREAM * KB
NBLK_E = 98
EPT = BLK * NBLK_E
EPAD = EPT * NS
EROWS = EPAD // STREAM

N_UP = 100096
ROWS_PT = N_UP // NS
ZR = 368

KD = 8
NBLK_D = 49
EPT_D = KD * STREAM * NBLK_D
DROWS_PT = N_UP // NS
ZB1 = 368

NPT = 8192
NODE_PAD = NPT * NS
NBLK_P = NPT // BLK
GP = G + 16
CNT = 96


def _fill_zero_2d(buf, nrows):
    def body(i, c):
        buf[i, :] = jnp.zeros((LANES,), jnp.float32)
        return c
    lax.fori_loop(0, nrows, body, 0)


def _fill_1d(buf, nvals, value):
    def body(i, c):
        buf[pl.ds(i * LANES, LANES)] = jnp.full((LANES,), value, jnp.float32)
        return c
    lax.fori_loop(0, nvals // LANES, body, 0)


def _deg_body(dstm, deg_a, deg_b, idx_d, ones_b, zb1, acc, ssem):
    c = lax.axis_index("c")
    s = lax.axis_index("s")
    _fill_1d(ones_b, STREAM, 1.0)
    _fill_1d(zb1, ZB1, 0.0)

    def zc(b, carry):
        pltpu.sync_copy(zb1, acc.at[pl.ds(s * DROWS_PT + b * ZB1, ZB1)])
        return carry
    lax.fori_loop(0, DROWS_PT // ZB1, zc, 0)
    plsc.subcore_barrier()

    row0 = c * (EPAD // 2 // STREAM) + s * (EPT_D // STREAM)

    def blk(b, carry):
        pltpu.sync_copy(dstm.at[pl.ds(row0 + b * KD, KD)], idx_d)
        descs = [pltpu.async_copy(ones_b, acc.at[idx_d.at[j]], ssem, add=True)
                 for j in range(KD)]
        for d in descs:
            d.wait()
        return carry
    lax.fori_loop(0, NBLK_D, blk, 0)
    plsc.subcore_barrier()

    def writeout(out):
        def cp(b, carry):
            off = s * DROWS_PT + b * ZB1
            pltpu.sync_copy(acc.at[pl.ds(off, ZB1)], zb1)
            pltpu.sync_copy(zb1, out.at[pl.ds(off, ZB1)])
            return carry
        lax.fori_loop(0, DROWS_PT // ZB1, cp, 0)

    @pl.when(c == 0)
    def _():
        writeout(deg_a)

    @pl.when(c == 1)
    def _():
        writeout(deg_b)


_deg_call = pl.kernel(
    _deg_body,
    out_type=[jax.ShapeDtypeStruct((N_UP,), jnp.float32),
              jax.ShapeDtypeStruct((N_UP,), jnp.float32)],
    mesh=plsc.VectorSubcoreMesh(core_axis_name="c", subcore_axis_name="s"),
    compiler_params=pltpu.CompilerParams(use_tc_tiling_on_sc=False),
    scratch_types=[
        pltpu.VMEM((KD, STREAM), jnp.int32),
        pltpu.VMEM((STREAM,), jnp.float32),
        pltpu.VMEM((ZB1,), jnp.float32),
        pltpu.VMEM_SHARED((N_UP,), jnp.float32),
        pltpu.SemaphoreType.DMA,
    ],
)


def _edge_body(hs_a, hs_b, srcm, dstm, agg_a, agg_b,
               idx_s, idx_d, rows, zbuf, acc, gsem, ssem):
    c = lax.axis_index("c")
    s = lax.axis_index("s")
    _fill_zero_2d(zbuf, ZR)

    def zc(b, carry):
        pltpu.sync_copy(zbuf, acc.at[pl.ds(s * ROWS_PT + b * ZR, ZR)])
        return carry
    lax.fori_loop(0, ROWS_PT // ZR, zc, 0)
    plsc.subcore_barrier()

    row0 = s * (EPT // STREAM)

    def pipeline(table, out):
        def blk(b, carry):
            rb = row0 + b * KB
            pltpu.sync_copy(srcm.at[pl.ds(rb, KB)], idx_s)
            pltpu.sync_copy(dstm.at[pl.ds(rb, KB)], idx_d)
            gd = [pltpu.async_copy(table.at[idx_s.at[j]], rows.at[j], gsem)
                  for j in range(KB)]
            for d in gd:
                d.wait()
            sd = [pltpu.async_copy(rows.at[j], acc.at[idx_d.at[j]], ssem,
                                   add=True)
                  for j in range(KB)]
            for d in sd:
                d.wait()
            return carry
        lax.fori_loop(0, NBLK_E, blk, 0)
        plsc.subcore_barrier()

        def cp(b, carry):
            off = s * ROWS_PT + b * ZR
            pltpu.sync_copy(acc.at[pl.ds(off, ZR)], zbuf)
            pltpu.sync_copy(zbuf, out.at[pl.ds(off, ZR)])
            return carry
        lax.fori_loop(0, ROWS_PT // ZR, cp, 0)

    @pl.when(c == 0)
    def _():
        pipeline(hs_a, agg_a)

    @pl.when(c == 1)
    def _():
        pipeline(hs_b, agg_b)


_edge_call = pl.kernel(
    _edge_body,
    out_type=[jax.ShapeDtypeStruct((N_UP, HH), jnp.float32),
              jax.ShapeDtypeStruct((N_UP, HH), jnp.float32)],
    mesh=plsc.VectorSubcoreMesh(core_axis_name="c", subcore_axis_name="s"),
    compiler_params=pltpu.CompilerParams(use_tc_tiling_on_sc=False),
    scratch_types=[
        pltpu.VMEM((KB, STREAM), jnp.int32),
        pltpu.VMEM((KB, STREAM), jnp.int32),
        pltpu.VMEM((KB, STREAM, HH), jnp.float32),
        pltpu.VMEM((ZR, HH), jnp.float32),
        pltpu.VMEM_SHARED((N_UP, HH), jnp.float32),
        pltpu.SemaphoreType.DMA,
        pltpu.SemaphoreType.DMA,
    ],
)


def _pool_body(h_a, h_b, nidm, batm, pooled_a, pooled_b, cnt,
               idx_s, idx_d, rows, zbuf, ones_b, cbuf, accp, accc, gsem, ssem):
    c = lax.axis_index("c")
    s = lax.axis_index("s")
    _fill_zero_2d(zbuf, ZR)
    _fill_1d(ones_b, STREAM, 1.0)
    pltpu.sync_copy(zbuf.at[pl.ds(0, GP // NS)],
                    accp.at[pl.ds(s * (GP // NS), GP // NS)])

    @pl.when((c == 0) & (s == 0))
    def _():
        _fill_1d(cbuf, CNT, 0.0)
        pltpu.sync_copy(cbuf, accc)
    plsc.subcore_barrier()

    row0 = s * (NPT // STREAM)

    def pipeline(table, want_cnt):
        def blk(b, carry):
            rb = row0 + b * KB
            pltpu.sync_copy(nidm.at[pl.ds(rb, KB)], idx_s)
            pltpu.sync_copy(batm.at[pl.ds(rb, KB)], idx_d)
            gd = [pltpu.async_copy(table.at[idx_s.at[j]], rows.at[j], gsem)
                  for j in range(KB)]
            for d in gd:
                d.wait()
            sd = [pltpu.async_copy(rows.at[j], accp.at[idx_d.at[j]], ssem,
                                   add=True)
                  for j in range(KB)]
            if want_cnt:
                sd += [pltpu.async_copy(ones_b, accc.at[idx_d.at[j]], ssem,
                                        add=True)
                       for j in range(KB)]
            for d in sd:
                d.wait()
            return carry
        lax.fori_loop(0, NBLK_P, blk, 0)

    @pl.when(c == 0)
    def _():
        pipeline(h_a, True)

    @pl.when(c == 1)
    def _():
        pipeline(h_b, False)
    plsc.subcore_barrier()

    @pl.when((c == 0) & (s == 0))
    def _():
        pltpu.sync_copy(accp, zbuf.at[pl.ds(0, GP)])
        pltpu.sync_copy(zbuf.at[pl.ds(0, GP)], pooled_a)
        pltpu.sync_copy(accc, cbuf)
        pltpu.sync_copy(cbuf, cnt)

    @pl.when((c == 1) & (s == 0))
    def _():
        pltpu.sync_copy(accp, zbuf.at[pl.ds(0, GP)])
        pltpu.sync_copy(zbuf.at[pl.ds(0, GP)], pooled_b)


_pool_call = pl.kernel(
    _pool_body,
    out_type=[jax.ShapeDtypeStruct((GP, HH), jnp.float32),
              jax.ShapeDtypeStruct((GP, HH), jnp.float32),
              jax.ShapeDtypeStruct((CNT,), jnp.float32)],
    mesh=plsc.VectorSubcoreMesh(core_axis_name="c", subcore_axis_name="s"),
    compiler_params=pltpu.CompilerParams(use_tc_tiling_on_sc=False),
    scratch_types=[
        pltpu.VMEM((KB, STREAM), jnp.int32),
        pltpu.VMEM((KB, STREAM), jnp.int32),
        pltpu.VMEM((KB, STREAM, HH), jnp.float32),
        pltpu.VMEM((ZR, HH), jnp.float32),
        pltpu.VMEM((STREAM,), jnp.float32),
        pltpu.VMEM((CNT,), jnp.float32),
        pltpu.VMEM_SHARED((GP, HH), jnp.float32),
        pltpu.VMEM_SHARED((CNT,), jnp.float32),
        pltpu.SemaphoreType.DMA,
        pltpu.SemaphoreType.DMA,
    ],
)


TB = 1000


def _tc0_body(x, da, db, ew, eb, dinv_o, hsa_o, hsb_o):
    deg = da[:] + db[:] + 1.0
    dinv = lax.rsqrt(deg)
    h0 = jnp.dot(x[:], ew[:], preferred_element_type=jnp.float32) + eb[:]
    hs = h0 * dinv
    dinv_o[:] = dinv
    hsa_o[:] = hs[:, :HH]
    hsb_o[:] = hs[:, HH:]


def _tc0(x, deg_a, deg_b, embed_W, embed_b):
    return pl.pallas_call(
        _tc0_body,
        grid=(N // TB,),
        in_specs=[
            pl.BlockSpec((TB, IN_F), lambda b: (b, 0)),
            pl.BlockSpec((TB, 1), lambda b: (b, 0)),
            pl.BlockSpec((TB, 1), lambda b: (b, 0)),
            pl.BlockSpec((IN_F, H), lambda b: (0, 0)),
            pl.BlockSpec((H,), lambda b: (0,)),
        ],
        out_specs=[
            pl.BlockSpec((TB, 1), lambda b: (b, 0)),
            pl.BlockSpec((TB, HH), lambda b: (b, 0)),
            pl.BlockSpec((TB, HH), lambda b: (b, 0)),
        ],
        out_shape=[
            jax.ShapeDtypeStruct((N, 1), jnp.float32),
            jax.ShapeDtypeStruct((N, HH), jnp.float32),
            jax.ShapeDtypeStruct((N, HH), jnp.float32),
        ],
    )(x, deg_a, deg_b, embed_W, embed_b)


def _layer_body(scale_out, aa, ab, ha, hb, dinv, w, bvec, oa, ob):
    dv = dinv[:]
    t = jnp.concatenate([aa[:] + ha[:], ab[:] + hb[:]], axis=1) * dv
    h = jnp.dot(t, w[:], preferred_element_type=jnp.float32) + bvec[:]
    h = jnp.maximum(h, 0.0)
    if scale_out:
        h = h * dv
    oa[:] = h[:, :HH]
    ob[:] = h[:, HH:]


def _tc_layer(agg_a, agg_b, hs_a, hs_b, dinv, w, bvec, scale_out):
    return pl.pallas_call(
        functools.partial(_layer_body, scale_out),
        grid=(N // TB,),
        in_specs=[
            pl.BlockSpec((TB, HH), lambda b: (b, 0)),
            pl.BlockSpec((TB, HH), lambda b: (b, 0)),
            pl.BlockSpec((TB, HH), lambda b: (b, 0)),
            pl.BlockSpec((TB, HH), lambda b: (b, 0)),
            pl.BlockSpec((TB, 1), lambda b: (b, 0)),
            pl.BlockSpec((H, H), lambda b: (0, 0)),
            pl.BlockSpec((H,), lambda b: (0,)),
        ],
        out_specs=[
            pl.BlockSpec((TB, HH), lambda b: (b, 0)),
            pl.BlockSpec((TB, HH), lambda b: (b, 0)),
        ],
        out_shape=[
            jax.ShapeDtypeStruct((N, HH), jnp.float32),
            jax.ShapeDtypeStruct((N, HH), jnp.float32),
        ],
    )(agg_a, agg_b, hs_a, hs_b, dinv, w, bvec)


def _final_body(pa, pb, cn, fw, fb, out):
    p = jnp.concatenate([pa[:G, :], pb[:G, :]], axis=1)
    mean = p / jnp.maximum(cn[:G], 1.0)[:, None]
    out[:] = jnp.dot(mean, fw[:], preferred_element_type=jnp.float32) + fb[:]


def _tc_final(pooled_a, pooled_b, cnt, fc_W, fc_b):
    return pl.pallas_call(
        _final_body,
        out_shape=jax.ShapeDtypeStruct((G, OUT_F), jnp.float32),
    )(pooled_a, pooled_b, cnt, fc_W, fc_b)


def kernel(x, edge_index, batch, embed_W, embed_b, conv_W, conv_b, fc_W, fc_b):
    src = edge_index[0]
    dst = edge_index[1]
    pad = EPAD - E
    srcm = jnp.concatenate(
        [src, jnp.zeros((pad,), jnp.int32)]).reshape(EROWS, STREAM)
    dstm = jnp.concatenate(
        [dst, jnp.full((pad,), N, jnp.int32)]).reshape(EROWS, STREAM)

    nidm = jnp.concatenate(
        [jnp.arange(N, dtype=jnp.int32),
         jnp.zeros((NODE_PAD - N,), jnp.int32)]).reshape(-1, STREAM)
    batm = jnp.concatenate(
        [batch, jnp.full((NODE_PAD - N,), G, jnp.int32)]).reshape(-1, STREAM)

    deg_a, deg_b = _deg_call(dstm)
    dinv, hs_a, hs_b = _tc0(x, deg_a[:N].reshape(N, 1),
                            deg_b[:N].reshape(N, 1), embed_W, embed_b)

    for i in range(L_LAYERS):
        agg_a, agg_b = _edge_call(hs_a, hs_b, srcm, dstm)
        hs_a, hs_b = _tc_layer(agg_a, agg_b, hs_a, hs_b, dinv,
                               conv_W[i], conv_b[i],
                               scale_out=(i < L_LAYERS - 1))

    pooled_a, pooled_b, cnt = _pool_call(hs_a, hs_b, nidm, batm)
    return _tc_final(pooled_a, pooled_b, cnt, fc_W, fc_b)


# speedup vs baseline: 18.9688x; 1.3286x over previous
import functools

import jax
import jax.numpy as jnp
from jax import lax
from jax.experimental import pallas as pl
from jax.experimental.pallas import tpu as pltpu
from jax.experimental.pallas import tpu_sc as plsc

N = 100000
E = 1600000
IN_F = 2
H = 32
HH = 16
L_LAYERS = 9
OUT_F = 2
G = 64

NC = 2
NS = 16
LANES = 16

STREAM = 128
KB = 4
BLK = STREAM * KB
NBLK_E = 196
EPT = BLK * NBLK_E
EPAD = EPT * NS
EROWS = EPAD // STREAM

N_UP = 100096
ROWS_PT = N_UP // NS
ZR = 368

KD = 8
NBLK_D = 49
EPT_D = KD * STREAM * NBLK_D
DROWS_PT = N_UP // NS
ZB1 = 368

NPT = 8192
NODE_PAD = NPT * NS
NBLK_P = NPT // BLK
GP = G + 16
CNT = 96


def _fill_zero_2d(buf, nrows):
    def body(i, c):
        buf[i, :] = jnp.zeros((LANES,), jnp.float32)
        return c
    lax.fori_loop(0, nrows, body, 0)


def _fill_1d(buf, nvals, value):
    def body(i, c):
        buf[pl.ds(i * LANES, LANES)] = jnp.full((LANES,), value, jnp.float32)
        return c
    lax.fori_loop(0, nvals // LANES, body, 0)


def _deg_body(edg, deg_a, deg_b, idx_d, ones_b, zb1, acc, ssem):
    c = lax.axis_index("c")
    s = lax.axis_index("s")
    _fill_1d(ones_b, STREAM, 1.0)
    _fill_1d(zb1, ZB1, 0.0)

    def zc(b, carry):
        pltpu.sync_copy(zb1, acc.at[pl.ds(s * DROWS_PT + b * ZB1, ZB1)])
        return carry
    lax.fori_loop(0, DROWS_PT // ZB1, zc, 0)
    plsc.subcore_barrier()

    row0 = c * (EPAD // 2 // STREAM) + s * (EPT_D // STREAM)

    def blk(b, carry):
        pltpu.sync_copy(edg.at[pl.ds(row0 + b * KD, KD)], idx_d)
        descs = [pltpu.async_copy(ones_b, acc.at[idx_d.at[j, 1]], ssem,
                                  add=True)
                 for j in range(KD)]
        for d in descs:
            d.wait()
        return carry
    lax.fori_loop(0, NBLK_D, blk, 0)
    plsc.subcore_barrier()

    def writeout(out):
        def cp(b, carry):
            off = s * DROWS_PT + b * ZB1
            pltpu.sync_copy(acc.at[pl.ds(off, ZB1)], zb1)
            pltpu.sync_copy(zb1, out.at[pl.ds(off, ZB1)])
            return carry
        lax.fori_loop(0, DROWS_PT // ZB1, cp, 0)

    @pl.when(c == 0)
    def _():
        writeout(deg_a)

    @pl.when(c == 1)
    def _():
        writeout(deg_b)


_deg_call = pl.kernel(
    _deg_body,
    out_type=[jax.ShapeDtypeStruct((N_UP,), jnp.float32),
              jax.ShapeDtypeStruct((N_UP,), jnp.float32)],
    mesh=plsc.VectorSubcoreMesh(core_axis_name="c", subcore_axis_name="s"),
    compiler_params=pltpu.CompilerParams(use_tc_tiling_on_sc=False),
    scratch_types=[
        pltpu.VMEM((KD, 2, STREAM), jnp.int32),
        pltpu.VMEM((STREAM,), jnp.float32),
        pltpu.VMEM((ZB1,), jnp.float32),
        pltpu.VMEM_SHARED((N_UP,), jnp.float32),
        pltpu.SemaphoreType.DMA,
    ],
)


def _edge_body(hs_a, hs_b, edg, agg_a, agg_b,
               cidx, rows, zbuf, acc, isem, gsem, ssem):
    c = lax.axis_index("c")
    s = lax.axis_index("s")
    _fill_zero_2d(zbuf, ZR)

    def zc(b, carry):
        pltpu.sync_copy(zbuf, acc.at[pl.ds(s * ROWS_PT + b * ZR, ZR)])
        return carry
    lax.fori_loop(0, ROWS_PT // ZR, zc, 0)
    plsc.subcore_barrier()

    rbase = s * (EPT // STREAM)

    def pipeline(table, out):
        for j in range(2):
            pltpu.async_copy(edg.at[pl.ds(rbase + j * KB, KB)],
                             cidx.at[pl.ds(j * KB, KB)], isem.at[j])

        def it(b, carry):
            r2 = lax.rem(b, 2)
            sl_b = lax.rem(b, 4) * KB
            sl_p = lax.rem(b + 2, 4) * KB

            @pl.when(b >= 2)
            def _():
                for j in range(KB):
                    pltpu.make_async_copy(
                        table.at[cidx.at[sl_p + j, 0]],
                        rows.at[r2 * KB + j], gsem.at[r2]).wait()
                sd = [pltpu.async_copy(rows.at[r2 * KB + j],
                                       acc.at[cidx.at[sl_p + j, 1]],
                                       ssem, add=True)
                      for j in range(KB)]
                for d in sd:
                    d.wait()

            @pl.when(b < NBLK_E)
            def _():
                pltpu.make_async_copy(
                    edg.at[pl.ds(rbase + b * KB, KB)],
                    cidx.at[pl.ds(sl_b, KB)], isem.at[r2]).wait()
                for j in range(KB):
                    pltpu.async_copy(table.at[cidx.at[sl_b + j, 0]],
                                     rows.at[r2 * KB + j], gsem.at[r2])

            @pl.when(b + 2 < NBLK_E)
            def _():
                pltpu.async_copy(edg.at[pl.ds(rbase + (b + 2) * KB, KB)],
                                 cidx.at[pl.ds(sl_p, KB)], isem.at[r2])
            return carry
        lax.fori_loop(0, NBLK_E + 2, it, 0)
        plsc.subcore_barrier()

        def cp(b, carry):
            off = s * ROWS_PT + b * ZR
            pltpu.sync_copy(acc.at[pl.ds(off, ZR)], zbuf)
            pltpu.sync_copy(zbuf, out.at[pl.ds(off, ZR)])
            return carry
        lax.fori_loop(0, ROWS_PT // ZR, cp, 0)

    @pl.when(c == 0)
    def _():
        pipeline(hs_a, agg_a)

    @pl.when(c == 1)
    def _():
        pipeline(hs_b, agg_b)


_edge_call = pl.kernel(
    _edge_body,
    out_type=[jax.ShapeDtypeStruct((N_UP, HH), jnp.float32),
              jax.ShapeDtypeStruct((N_UP, HH), jnp.float32)],
    mesh=plsc.VectorSubcoreMesh(core_axis_name="c", subcore_axis_name="s"),
    compiler_params=pltpu.CompilerParams(use_tc_tiling_on_sc=False),
    scratch_types=[
        pltpu.VMEM((4 * KB, 2, STREAM), jnp.int32),
        pltpu.VMEM((2 * KB, STREAM, HH), jnp.float32),
        pltpu.VMEM((ZR, HH), jnp.float32),
        pltpu.VMEM_SHARED((N_UP, HH), jnp.float32),
        pltpu.SemaphoreType.DMA((2,)),
        pltpu.SemaphoreType.DMA((2,)),
        pltpu.SemaphoreType.DMA,
    ],
)


def _pool_body(h_a, h_b, pidm, pooled_a, pooled_b, cnt,
               cid, rows, zbuf, ones_b, cbuf, accp, accc, gsem, ssem):
    c = lax.axis_index("c")
    s = lax.axis_index("s")
    _fill_zero_2d(zbuf, ZR)
    _fill_1d(ones_b, STREAM, 1.0)
    pltpu.sync_copy(zbuf.at[pl.ds(0, GP // NS)],
                    accp.at[pl.ds(s * (GP // NS), GP // NS)])

    @pl.when((c == 0) & (s == 0))
    def _():
        _fill_1d(cbuf, CNT, 0.0)
        pltpu.sync_copy(cbuf, accc)
    plsc.subcore_barrier()

    row0 = s * (NPT // STREAM)

    def pipeline(table, want_cnt):
        def blk(b, carry):
            rb = row0 + b * KB
            pltpu.sync_copy(pidm.at[pl.ds(rb, KB)], cid)
            gd = [pltpu.async_copy(table.at[cid.at[j, 0]], rows.at[j], gsem)
                  for j in range(KB)]
            for d in gd:
                d.wait()
            sd = [pltpu.async_copy(rows.at[j], accp.at[cid.at[j, 1]], ssem,
                                   add=True)
                  for j in range(KB)]
            if want_cnt:
                sd += [pltpu.async_copy(ones_b, accc.at[cid.at[j, 1]], ssem,
                                        add=True)
                       for j in range(KB)]
            for d in sd:
                d.wait()
            return carry
        lax.fori_loop(0, NBLK_P, blk, 0)

    @pl.when(c == 0)
    def _():
        pipeline(h_a, True)

    @pl.when(c == 1)
    def _():
        pipeline(h_b, False)
    plsc.subcore_barrier()

    @pl.when((c == 0) & (s == 0))
    def _():
        pltpu.sync_copy(accp, zbuf.at[pl.ds(0, GP)])
        pltpu.sync_copy(zbuf.at[pl.ds(0, GP)], pooled_a)
        pltpu.sync_copy(accc, cbuf)
        pltpu.sync_copy(cbuf, cnt)

    @pl.when((c == 1) & (s == 0))
    def _():
        pltpu.sync_copy(accp, zbuf.at[pl.ds(0, GP)])
        pltpu.sync_copy(zbuf.at[pl.ds(0, GP)], pooled_b)


_pool_call = pl.kernel(
    _pool_body,
    out_type=[jax.ShapeDtypeStruct((GP, HH), jnp.float32),
              jax.ShapeDtypeStruct((GP, HH), jnp.float32),
              jax.ShapeDtypeStruct((CNT,), jnp.float32)],
    mesh=plsc.VectorSubcoreMesh(core_axis_name="c", subcore_axis_name="s"),
    compiler_params=pltpu.CompilerParams(use_tc_tiling_on_sc=False),
    scratch_types=[
        pltpu.VMEM((KB, 2, STREAM), jnp.int32),
        pltpu.VMEM((KB, STREAM, HH), jnp.float32),
        pltpu.VMEM((ZR, HH), jnp.float32),
        pltpu.VMEM((STREAM,), jnp.float32),
        pltpu.VMEM((CNT,), jnp.float32),
        pltpu.VMEM_SHARED((GP, HH), jnp.float32),
        pltpu.VMEM_SHARED((CNT,), jnp.float32),
        pltpu.SemaphoreType.DMA,
        pltpu.SemaphoreType.DMA,
    ],
)


TB = 1000


def _tc0_body(x, da, db, ew, eb, dinv_o, hsa_o, hsb_o):
    deg = da[:] + db[:] + 1.0
    dinv = lax.rsqrt(deg)
    h0 = jnp.dot(x[:], ew[:], preferred_element_type=jnp.float32) + eb[:]
    hs = h0 * dinv
    dinv_o[:] = dinv
    hsa_o[:] = hs[:, :HH]
    hsb_o[:] = hs[:, HH:]


def _tc0(x, deg_a, deg_b, embed_W, embed_b):
    return pl.pallas_call(
        _tc0_body,
        grid=(N // TB,),
        in_specs=[
            pl.BlockSpec((TB, IN_F), lambda b: (b, 0)),
            pl.BlockSpec((TB, 1), lambda b: (b, 0)),
            pl.BlockSpec((TB, 1), lambda b: (b, 0)),
            pl.BlockSpec((IN_F, H), lambda b: (0, 0)),
            pl.BlockSpec((H,), lambda b: (0,)),
        ],
        out_specs=[
            pl.BlockSpec((TB, 1), lambda b: (b, 0)),
            pl.BlockSpec((TB, HH), lambda b: (b, 0)),
            pl.BlockSpec((TB, HH), lambda b: (b, 0)),
        ],
        out_shape=[
            jax.ShapeDtypeStruct((N, 1), jnp.float32),
            jax.ShapeDtypeStruct((N, HH), jnp.float32),
            jax.ShapeDtypeStruct((N, HH), jnp.float32),
        ],
    )(x, deg_a, deg_b, embed_W, embed_b)


def _layer_body(scale_out, aa, ab, ha, hb, dinv, w, bvec, oa, ob):
    dv = dinv[:]
    t = jnp.concatenate([aa[:] + ha[:], ab[:] + hb[:]], axis=1) * dv
    h = jnp.dot(t, w[:], preferred_element_type=jnp.float32) + bvec[:]
    h = jnp.maximum(h, 0.0)
    if scale_out:
        h = h * dv
    oa[:] = h[:, :HH]
    ob[:] = h[:, HH:]


def _tc_layer(agg_a, agg_b, hs_a, hs_b, dinv, w, bvec, scale_out):
    return pl.pallas_call(
        functools.partial(_layer_body, scale_out),
        grid=(N // TB,),
        in_specs=[
            pl.BlockSpec((TB, HH), lambda b: (b, 0)),
            pl.BlockSpec((TB, HH), lambda b: (b, 0)),
            pl.BlockSpec((TB, HH), lambda b: (b, 0)),
            pl.BlockSpec((TB, HH), lambda b: (b, 0)),
            pl.BlockSpec((TB, 1), lambda b: (b, 0)),
            pl.BlockSpec((H, H), lambda b: (0, 0)),
            pl.BlockSpec((H,), lambda b: (0,)),
        ],
        out_specs=[
            pl.BlockSpec((TB, HH), lambda b: (b, 0)),
            pl.BlockSpec((TB, HH), lambda b: (b, 0)),
        ],
        out_shape=[
            jax.ShapeDtypeStruct((N, HH), jnp.float32),
            jax.ShapeDtypeStruct((N, HH), jnp.float32),
        ],
    )(agg_a, agg_b, hs_a, hs_b, dinv, w, bvec)


def _final_body(pa, pb, cn, fw, fb, out):
    p = jnp.concatenate([pa[:G, :], pb[:G, :]], axis=1)
    mean = p / jnp.maximum(cn[:G], 1.0)[:, None]
    out[:] = jnp.dot(mean, fw[:], preferred_element_type=jnp.float32) + fb[:]


def _tc_final(pooled_a, pooled_b, cnt, fc_W, fc_b):
    return pl.pallas_call(
        _final_body,
        out_shape=jax.ShapeDtypeStruct((G, OUT_F), jnp.float32),
    )(pooled_a, pooled_b, cnt, fc_W, fc_b)


def kernel(x, edge_index, batch, embed_W, embed_b, conv_W, conv_b, fc_W, fc_b):
    src = edge_index[0]
    dst = edge_index[1]
    pad = EPAD - E
    srcm = jnp.concatenate(
        [src, jnp.zeros((pad,), jnp.int32)]).reshape(EROWS, 1, STREAM)
    dstm = jnp.concatenate(
        [dst, jnp.full((pad,), N, jnp.int32)]).reshape(EROWS, 1, STREAM)
    edg = jnp.concatenate([srcm, dstm], axis=1)

    nidm = jnp.concatenate(
        [jnp.arange(N, dtype=jnp.int32),
         jnp.zeros((NODE_PAD - N,), jnp.int32)]).reshape(-1, 1, STREAM)
    batm = jnp.concatenate(
        [batch, jnp.full((NODE_PAD - N,), G, jnp.int32)]).reshape(-1, 1, STREAM)
    pidm = jnp.concatenate([nidm, batm], axis=1)

    deg_a, deg_b = _deg_call(edg)
    dinv, hs_a, hs_b = _tc0(x, deg_a[:N].reshape(N, 1),
                            deg_b[:N].reshape(N, 1), embed_W, embed_b)

    for i in range(L_LAYERS):
        agg_a, agg_b = _edge_call(hs_a, hs_b, edg)
        hs_a, hs_b = _tc_layer(agg_a, agg_b, hs_a, hs_b, dinv,
                               conv_W[i], conv_b[i],
                               scale_out=(i < L_LAYERS - 1))

    pooled_a, pooled_b, cnt = _pool_call(hs_a, hs_b, pidm)
    return _tc_final(pooled_a, pooled_b, cnt, fc_W, fc_b)


# speedup vs baseline: 31.9018x; 1.6818x over previous
import functools

import jax
import jax.numpy as jnp
from jax import lax
from jax.experimental import pallas as pl
from jax.experimental.pallas import tpu as pltpu
from jax.experimental.pallas import tpu_sc as plsc

N = 100000
E = 1600000
IN_F = 2
H = 32
HH = 16
L_LAYERS = 9
OUT_F = 2
G = 64

NC = 2
NS = 16
LANES = 16

STREAM = 128
KB = 2
BLK = STREAM * KB
NBLK_E = 392
EPT = BLK * NBLK_E
EPAD = EPT * NS
EROWS = EPAD // STREAM

N_UP = 100352
ROWS_PT = N_UP // NS
ZR = 784

KD = 8
NBLK_D = 49
EPT_D = KD * STREAM * NBLK_D
DROWS_PT = N_UP // NS
ZB1 = 784

NPT = 8192
NODE_PAD = NPT * NS
NBLK_P = NPT // BLK
GP = G + 16
CNT = 96


def _fill_zero_2d(buf, nrows):
    def body(i, c):
        buf[i, :] = jnp.zeros((LANES,), jnp.float32)
        return c
    lax.fori_loop(0, nrows, body, 0)


def _fill_1d(buf, nvals, value):
    def body(i, c):
        buf[pl.ds(i * LANES, LANES)] = jnp.full((LANES,), value, jnp.float32)
        return c
    lax.fori_loop(0, nvals // LANES, body, 0)


def _deg_body(edg, deg_a, deg_b, idx_d, ones_b, zb1, acc, ssem):
    c = lax.axis_index("c")
    s = lax.axis_index("s")
    _fill_1d(ones_b, STREAM, 1.0)
    _fill_1d(zb1, ZB1, 0.0)

    def zc(b, carry):
        pltpu.sync_copy(zb1, acc.at[pl.ds(s * DROWS_PT + b * ZB1, ZB1)])
        return carry
    lax.fori_loop(0, DROWS_PT // ZB1, zc, 0)
    plsc.subcore_barrier()

    row0 = c * (EPAD // 2 // STREAM) + s * (EPT_D // STREAM)

    def blk(b, carry):
        pltpu.sync_copy(edg.at[pl.ds(row0 + b * KD, KD)], idx_d)
        descs = [pltpu.async_copy(ones_b, acc.at[idx_d.at[j, 1]], ssem,
                                  add=True)
                 for j in range(KD)]
        for d in descs:
            d.wait()
        return carry
    lax.fori_loop(0, NBLK_D, blk, 0)
    plsc.subcore_barrier()

    def writeout(out):
        def cp(b, carry):
            off = s * DROWS_PT + b * ZB1
            pltpu.sync_copy(acc.at[pl.ds(off, ZB1)], zb1)
            pltpu.sync_copy(zb1, out.at[pl.ds(off, ZB1)])
            return carry
        lax.fori_loop(0, DROWS_PT // ZB1, cp, 0)

    @pl.when(c == 0)
    def _():
        writeout(deg_a)

    @pl.when(c == 1)
    def _():
        writeout(deg_b)


_deg_call = pl.kernel(
    _deg_body,
    out_type=[jax.ShapeDtypeStruct((N_UP,), jnp.float32),
              jax.ShapeDtypeStruct((N_UP,), jnp.float32)],
    mesh=plsc.VectorSubcoreMesh(core_axis_name="c", subcore_axis_name="s"),
    compiler_params=pltpu.CompilerParams(use_tc_tiling_on_sc=False),
    scratch_types=[
        pltpu.VMEM((KD, 2, STREAM), jnp.int32),
        pltpu.VMEM((STREAM,), jnp.float32),
        pltpu.VMEM((ZB1,), jnp.float32),
        pltpu.VMEM_SHARED((N_UP,), jnp.float32),
        pltpu.SemaphoreType.DMA,
    ],
)


def _edge_body(hs_a, hs_b, edg, agg_a, agg_b,
               cidx, rows, zbuf, acc, isem, gsem, ssem):
    c = lax.axis_index("c")
    s = lax.axis_index("s")
    _fill_zero_2d(zbuf, ZR)

    def zc(b, carry):
        pltpu.sync_copy(zbuf, acc.at[pl.ds(s * ROWS_PT + b * ZR, ZR)])
        return carry
    lax.fori_loop(0, ROWS_PT // ZR, zc, 0)
    plsc.subcore_barrier()

    rbase = s * (EPT // STREAM)

    def pipeline(table, out):
        for j in range(2):
            pltpu.async_copy(edg.at[pl.ds(rbase + j * KB, KB)],
                             cidx.at[pl.ds(j * KB, KB)], isem.at[j])

        def it(b, carry):
            r2 = lax.rem(b, 2)
            sl_b = lax.rem(b, 4) * KB
            sl_p = lax.rem(b + 2, 4) * KB

            @pl.when(b >= 2)
            def _():
                for j in range(KB):
                    pltpu.make_async_copy(
                        table.at[cidx.at[sl_p + j, 0]],
                        rows.at[r2 * KB + j], gsem.at[r2]).wait()
                sd = [pltpu.async_copy(rows.at[r2 * KB + j],
                                       acc.at[cidx.at[sl_p + j, 1]],
                                       ssem, add=True)
                      for j in range(KB)]
                for d in sd:
                    d.wait()

            @pl.when(b < NBLK_E)
            def _():
                pltpu.make_async_copy(
                    edg.at[pl.ds(rbase + b * KB, KB)],
                    cidx.at[pl.ds(sl_b, KB)], isem.at[r2]).wait()
                for j in range(KB):
                    pltpu.async_copy(table.at[cidx.at[sl_b + j, 0]],
                                     rows.at[r2 * KB + j], gsem.at[r2])

            @pl.when(b + 2 < NBLK_E)
            def _():
                pltpu.async_copy(edg.at[pl.ds(rbase + (b + 2) * KB, KB)],
                                 cidx.at[pl.ds(sl_p, KB)], isem.at[r2])
            return carry
        lax.fori_loop(0, NBLK_E + 2, it, 0)
        plsc.subcore_barrier()

        def cp(b, carry):
            off = s * ROWS_PT + b * ZR
            pltpu.sync_copy(acc.at[pl.ds(off, ZR)], zbuf)
            pltpu.sync_copy(zbuf, out.at[pl.ds(off, ZR)])
            return carry
        lax.fori_loop(0, ROWS_PT // ZR, cp, 0)

    @pl.when(c == 0)
    def _():
        pipeline(hs_a, agg_a)

    @pl.when(c == 1)
    def _():
        pipeline(hs_b, agg_b)


_edge_call = pl.kernel(
    _edge_body,
    out_type=[jax.ShapeDtypeStruct((N_UP, HH), jnp.float32),
              jax.ShapeDtypeStruct((N_UP, HH), jnp.float32)],
    mesh=plsc.VectorSubcoreMesh(core_axis_name="c", subcore_axis_name="s"),
    compiler_params=pltpu.CompilerParams(use_tc_tiling_on_sc=False),
    scratch_types=[
        pltpu.VMEM((4 * KB, 2, STREAM), jnp.int32),
        pltpu.VMEM((2 * KB, STREAM, HH), jnp.float32),
        pltpu.VMEM((ZR, HH), jnp.float32),
        pltpu.VMEM_SHARED((N_UP, HH), jnp.float32),
        pltpu.SemaphoreType.DMA((2,)),
        pltpu.SemaphoreType.DMA((2,)),
        pltpu.SemaphoreType.DMA,
    ],
)


def _pool_body(h_a, h_b, pidm, pooled_a, pooled_b, cnt,
               cid, rows, zbuf, ones_b, cbuf, accp, accc, gsem, ssem):
    c = lax.axis_index("c")
    s = lax.axis_index("s")
    _fill_zero_2d(zbuf, ZR)
    _fill_1d(ones_b, STREAM, 1.0)
    pltpu.sync_copy(zbuf.at[pl.ds(0, GP // NS)],
                    accp.at[pl.ds(s * (GP // NS), GP // NS)])

    @pl.when((c == 0) & (s == 0))
    def _():
        _fill_1d(cbuf, CNT, 0.0)
        pltpu.sync_copy(cbuf, accc)
    plsc.subcore_barrier()

    row0 = s * (NPT // STREAM)

    def pipeline(table, want_cnt):
        def blk(b, carry):
            rb = row0 + b * KB
            pltpu.sync_copy(pidm.at[pl.ds(rb, KB)], cid)
            gd = [pltpu.async_copy(table.at[cid.at[j, 0]], rows.at[j], gsem)
                  for j in range(KB)]
            for d in gd:
                d.wait()
            sd = [pltpu.async_copy(rows.at[j], accp.at[cid.at[j, 1]], ssem,
                                   add=True)
                  for j in range(KB)]
            if want_cnt:
                sd += [pltpu.async_copy(ones_b, accc.at[cid.at[j, 1]], ssem,
                                        add=True)
                       for j in range(KB)]
            for d in sd:
                d.wait()
            return carry
        lax.fori_loop(0, NBLK_P, blk, 0)

    @pl.when(c == 0)
    def _():
        pipeline(h_a, True)

    @pl.when(c == 1)
    def _():
        pipeline(h_b, False)
    plsc.subcore_barrier()

    @pl.when((c == 0) & (s == 0))
    def _():
        pltpu.sync_copy(accp, zbuf.at[pl.ds(0, GP)])
        pltpu.sync_copy(zbuf.at[pl.ds(0, GP)], pooled_a)
        pltpu.sync_copy(accc, cbuf)
        pltpu.sync_copy(cbuf, cnt)

    @pl.when((c == 1) & (s == 0))
    def _():
        pltpu.sync_copy(accp, zbuf.at[pl.ds(0, GP)])
        pltpu.sync_copy(zbuf.at[pl.ds(0, GP)], pooled_b)


_pool_call = pl.kernel(
    _pool_body,
    out_type=[jax.ShapeDtypeStruct((GP, HH), jnp.float32),
              jax.ShapeDtypeStruct((GP, HH), jnp.float32),
              jax.ShapeDtypeStruct((CNT,), jnp.float32)],
    mesh=plsc.VectorSubcoreMesh(core_axis_name="c", subcore_axis_name="s"),
    compiler_params=pltpu.CompilerParams(use_tc_tiling_on_sc=False),
    scratch_types=[
        pltpu.VMEM((KB, 2, STREAM), jnp.int32),
        pltpu.VMEM((KB, STREAM, HH), jnp.float32),
        pltpu.VMEM((ZR, HH), jnp.float32),
        pltpu.VMEM((STREAM,), jnp.float32),
        pltpu.VMEM((CNT,), jnp.float32),
        pltpu.VMEM_SHARED((GP, HH), jnp.float32),
        pltpu.VMEM_SHARED((CNT,), jnp.float32),
        pltpu.SemaphoreType.DMA,
        pltpu.SemaphoreType.DMA,
    ],
)


NP128 = N_UP * HH // 128
BS0 = 112
TBP = 1568


def _tc0_body(xp, da, db, bde_a, bde_b, eb_a, eb_b, dinvp_o, hsa_o, hsb_o):
    deg = da[:] + db[:] + 1.0
    dvs = lax.rsqrt(deg)
    dp = jnp.broadcast_to(dvs.reshape(BS0, 16, 8, 1),
                          (BS0, 16, 8, 16)).reshape(BS0 * 16, 128)
    h0a = jnp.dot(xp[:], bde_a[:], preferred_element_type=jnp.float32) + eb_a[:]
    h0b = jnp.dot(xp[:], bde_b[:], preferred_element_type=jnp.float32) + eb_b[:]
    dinvp_o[:] = dp
    hsa_o[:] = h0a * dp
    hsb_o[:] = h0b * dp


def _tc0(xp, da, db, bde_a, bde_b, eb_a, eb_b):
    return pl.pallas_call(
        _tc0_body,
        grid=(NP128 // (16 * BS0),),
        in_specs=[
            pl.BlockSpec((16 * BS0, 2 * 8), lambda b: (b, 0)),
            pl.BlockSpec((BS0, 128), lambda b: (b, 0)),
            pl.BlockSpec((BS0, 128), lambda b: (b, 0)),
            pl.BlockSpec((2 * 8, 128), lambda b: (0, 0)),
            pl.BlockSpec((2 * 8, 128), lambda b: (0, 0)),
            pl.BlockSpec((128,), lambda b: (0,)),
            pl.BlockSpec((128,), lambda b: (0,)),
        ],
        out_specs=[
            pl.BlockSpec((16 * BS0, 128), lambda b: (b, 0)),
            pl.BlockSpec((16 * BS0, 128), lambda b: (b, 0)),
            pl.BlockSpec((16 * BS0, 128), lambda b: (b, 0)),
        ],
        out_shape=[
            jax.ShapeDtypeStruct((NP128, 128), jnp.float32),
            jax.ShapeDtypeStruct((NP128, 128), jnp.float32),
            jax.ShapeDtypeStruct((NP128, 128), jnp.float32),
        ],
    )(xp, da, db, bde_a, bde_b, eb_a, eb_b)


def _layer_body(scale_out, aa, ab, ha, hb, dp, waa, wab, wba, wbb,
                ba, bb2, oa, ob):
    dv = dp[:]
    ta = (aa[:] + ha[:]) * dv
    tb = (ab[:] + hb[:]) * dv
    hra = jnp.dot(ta, waa[:], preferred_element_type=jnp.float32)
    hra = hra + jnp.dot(tb, wba[:], preferred_element_type=jnp.float32)
    hra = jnp.maximum(hra + ba[:], 0.0)
    hrb = jnp.dot(ta, wab[:], preferred_element_type=jnp.float32)
    hrb = hrb + jnp.dot(tb, wbb[:], preferred_element_type=jnp.float32)
    hrb = jnp.maximum(hrb + bb2[:], 0.0)
    if scale_out:
        hra = hra * dv
        hrb = hrb * dv
    oa[:] = hra
    ob[:] = hrb


def _tc_layer(agg_a, agg_b, hs_a, hs_b, dinv_p, waa, wab, wba, wbb,
              ba, bb2, scale_out):
    blk = pl.BlockSpec((TBP, 128), lambda b: (b, 0))
    wblk = pl.BlockSpec((128, 128), lambda b: (0, 0))
    return pl.pallas_call(
        functools.partial(_layer_body, scale_out),
        grid=(NP128 // TBP,),
        in_specs=[blk, blk, blk, blk, blk, wblk, wblk, wblk, wblk,
                  pl.BlockSpec((128,), lambda b: (0,)),
                  pl.BlockSpec((128,), lambda b: (0,))],
        out_specs=[blk, blk],
        out_shape=[
            jax.ShapeDtypeStruct((NP128, 128), jnp.float32),
            jax.ShapeDtypeStruct((NP128, 128), jnp.float32),
        ],
    )(agg_a, agg_b, hs_a, hs_b, dinv_p, waa, wab, wba, wbb, ba, bb2)


def _final_body(pa, pb, cn, fw, fb, out):
    p = jnp.concatenate([pa[:G, :], pb[:G, :]], axis=1)
    mean = p / jnp.maximum(cn[:G], 1.0)[:, None]
    out[:] = jnp.dot(mean, fw[:], preferred_element_type=jnp.float32) + fb[:]


def _tc_final(pooled_a, pooled_b, cnt, fc_W, fc_b):
    return pl.pallas_call(
        _final_body,
        out_shape=jax.ShapeDtypeStruct((G, OUT_F), jnp.float32),
    )(pooled_a, pooled_b, cnt, fc_W, fc_b)


def kernel(x, edge_index, batch, embed_W, embed_b, conv_W, conv_b, fc_W, fc_b):
    src = edge_index[0]
    dst = edge_index[1]
    pad = EPAD - E
    srcm = jnp.concatenate(
        [src, jnp.zeros((pad,), jnp.int32)]).reshape(EROWS, 1, STREAM)
    dstm = jnp.concatenate(
        [dst, jnp.full((pad,), N, jnp.int32)]).reshape(EROWS, 1, STREAM)
    edg = jnp.concatenate([srcm, dstm], axis=1)

    nidm = jnp.concatenate(
        [jnp.arange(N, dtype=jnp.int32),
         jnp.zeros((NODE_PAD - N,), jnp.int32)]).reshape(-1, 1, STREAM)
    batm = jnp.concatenate(
        [batch, jnp.full((NODE_PAD - N,), G, jnp.int32)]).reshape(-1, 1, STREAM)
    pidm = jnp.concatenate([nidm, batm], axis=1)

    eye8 = jnp.eye(8, dtype=jnp.float32)
    bde_a = jnp.kron(eye8, embed_W[:, :HH])
    bde_b = jnp.kron(eye8, embed_W[:, HH:])
    eb_a = jnp.tile(embed_b[:HH], 8)
    eb_b = jnp.tile(embed_b[HH:], 8)

    deg_a, deg_b = _deg_call(edg)
    dinv_p, hs_a, hs_b = _tc0(x.reshape(N // 8, 16),
                              deg_a.reshape(N_UP // 128, 128),
                              deg_b.reshape(N_UP // 128, 128),
                              bde_a, bde_b, eb_a, eb_b)

    for i in range(L_LAYERS):
        wi = conv_W[i]
        waa = jnp.kron(eye8, wi[:HH, :HH])
        wab = jnp.kron(eye8, wi[:HH, HH:])
        wba = jnp.kron(eye8, wi[HH:, :HH])
        wbb = jnp.kron(eye8, wi[HH:, HH:])
        ba = jnp.tile(conv_b[i][:HH], 8)
        bb2 = jnp.tile(conv_b[i][HH:], 8)
        agg_a, agg_b = _edge_call(hs_a.reshape(N_UP, HH),
                                  hs_b.reshape(N_UP, HH), edg)
        hs_a, hs_b = _tc_layer(agg_a.reshape(NP128, 128),
                               agg_b.reshape(NP128, 128),
                               hs_a, hs_b, dinv_p, waa, wab, wba, wbb,
                               ba, bb2, scale_out=(i < L_LAYERS - 1))

    pooled_a, pooled_b, cnt = _pool_call(hs_a.reshape(N_UP, HH),
                                         hs_b.reshape(N_UP, HH), pidm)
    return _tc_final(pooled_a, pooled_b, cnt, fc_W, fc_b)


# speedup vs baseline: 37.2877x; 1.1688x over previous
import functools

import jax
import jax.numpy as jnp
from jax import lax
from jax.experimental import pallas as pl
from jax.experimental.pallas import tpu as pltpu
from jax.experimental.pallas import tpu_sc as plsc

N = 100000
E = 1600000
IN_F = 2
H = 32
HH = 16
L_LAYERS = 9
OUT_F = 2
G = 64

NC = 2
NS = 16
LANES = 16

STREAM = 128
KB = 4
BLK = STREAM * KB
NBLK_E = 196
EPT = BLK * NBLK_E
EPAD = EPT * NS
EROWS = EPAD // STREAM

N_UP = 100224
ROWS_PT = N_UP // NS
ZR = 232

KD = 8
NBLK_D = 49
EPT_D = KD * STREAM * NBLK_D
DROWS_PT = N_UP // NS
ZB1 = 232

NPT = 8192
NODE_PAD = NPT * NS
NBLK_P = NPT // BLK
GP = G + 16
CNT = 96


def _fill_zero_2d(buf, nrows):
    def body(i, c):
        buf[i, :] = jnp.zeros((LANES,), jnp.float32)
        return c
    lax.fori_loop(0, nrows, body, 0)


def _fill_1d(buf, nvals, value):
    def body(i, c):
        buf[pl.ds(i * LANES, LANES)] = jnp.full((LANES,), value, jnp.float32)
        return c
    lax.fori_loop(0, nvals // LANES, body, 0)


def _deg_body(edg, deg_a, deg_b, idx_d, ones_b, zb1, acc, ssem):
    c = lax.axis_index("c")
    s = lax.axis_index("s")
    _fill_1d(ones_b, STREAM, 1.0)
    _fill_1d(zb1, 240, 0.0)

    def zc(b, carry):
        pltpu.sync_copy(zb1.at[pl.ds(0, ZB1)],
                        acc.at[pl.ds(s * DROWS_PT + b * ZB1, ZB1)])
        return carry
    lax.fori_loop(0, DROWS_PT // ZB1, zc, 0)
    plsc.subcore_barrier()

    row0 = c * (EPAD // 2 // STREAM) + s * (EPT_D // STREAM)

    def blk(b, carry):
        pltpu.sync_copy(edg.at[pl.ds(row0 + b * KD, KD)], idx_d)
        descs = [pltpu.async_copy(ones_b, acc.at[idx_d.at[j, 1]], ssem,
                                  add=True)
                 for j in range(KD)]
        for d in descs:
            d.wait()
        return carry
    lax.fori_loop(0, NBLK_D, blk, 0)
    plsc.subcore_barrier()

    def writeout(out):
        def cp(b, carry):
            off = s * DROWS_PT + b * ZB1
            pltpu.sync_copy(acc.at[pl.ds(off, ZB1)], zb1.at[pl.ds(0, ZB1)])
            pltpu.sync_copy(zb1.at[pl.ds(0, ZB1)], out.at[pl.ds(off, ZB1)])
            return carry
        lax.fori_loop(0, DROWS_PT // ZB1, cp, 0)

    @pl.when(c == 0)
    def _():
        writeout(deg_a)

    @pl.when(c == 1)
    def _():
        writeout(deg_b)


_deg_call = pl.kernel(
    _deg_body,
    out_type=[jax.ShapeDtypeStruct((N_UP,), jnp.float32),
              jax.ShapeDtypeStruct((N_UP,), jnp.float32)],
    mesh=plsc.VectorSubcoreMesh(core_axis_name="c", subcore_axis_name="s"),
    compiler_params=pltpu.CompilerParams(use_tc_tiling_on_sc=False),
    scratch_types=[
        pltpu.VMEM((KD, 2, STREAM), jnp.int32),
        pltpu.VMEM((STREAM,), jnp.float32),
        pltpu.VMEM((240,), jnp.float32),
        pltpu.VMEM_SHARED((N_UP,), jnp.float32),
        pltpu.SemaphoreType.DMA,
    ],
)


def _edge_body(hs_a, hs_b, edg, agg_a, agg_b,
               cidx, rows, zbuf, acc, isem, gsem, ssem):
    c = lax.axis_index("c")
    s = lax.axis_index("s")
    _fill_zero_2d(zbuf, ZR)

    def zc(b, carry):
        pltpu.sync_copy(zbuf, acc.at[pl.ds(s * ROWS_PT + b * ZR, ZR)])
        return carry
    lax.fori_loop(0, ROWS_PT // ZR, zc, 0)
    plsc.subcore_barrier()

    rbase = s * (EPT // STREAM)

    def pipeline(table, out):
        for j in range(2):
            pltpu.async_copy(edg.at[pl.ds(rbase + j * KB, KB)],
                             cidx.at[pl.ds(j * KB, KB)], isem.at[j])

        def it(b, carry):
            r2 = lax.rem(b, 2)
            sl_b = lax.rem(b, 4) * KB
            sl_p = lax.rem(b + 2, 4) * KB

            @pl.when(b >= 2)
            def _():
                for j in range(KB):
                    pltpu.make_async_copy(
                        table.at[cidx.at[sl_p + j, 0]],
                        rows.at[r2 * KB + j], gsem.at[r2]).wait()
                sd = [pltpu.async_copy(rows.at[r2 * KB + j],
                                       acc.at[cidx.at[sl_p + j, 1]],
                                       ssem, add=True)
                      for j in range(KB)]
                for d in sd:
                    d.wait()

            @pl.when(b < NBLK_E)
            def _():
                pltpu.make_async_copy(
                    edg.at[pl.ds(rbase + b * KB, KB)],
                    cidx.at[pl.ds(sl_b, KB)], isem.at[r2]).wait()
                for j in range(KB):
                    pltpu.async_copy(table.at[cidx.at[sl_b + j, 0]],
                                     rows.at[r2 * KB + j], gsem.at[r2])

            @pl.when(b + 2 < NBLK_E)
            def _():
                pltpu.async_copy(edg.at[pl.ds(rbase + (b + 2) * KB, KB)],
                                 cidx.at[pl.ds(sl_p, KB)], isem.at[r2])
            return carry
        lax.fori_loop(0, NBLK_E + 2, it, 0)
        plsc.subcore_barrier()

        def cp(b, carry):
            off = s * ROWS_PT + b * ZR
            pltpu.sync_copy(acc.at[pl.ds(off, ZR)], zbuf)
            pltpu.sync_copy(zbuf, out.at[pl.ds(off, ZR)])
            return carry
        lax.fori_loop(0, ROWS_PT // ZR, cp, 0)

    @pl.when(c == 0)
    def _():
        pipeline(hs_a, agg_a)

    @pl.when(c == 1)
    def _():
        pipeline(hs_b, agg_b)


_edge_call = pl.kernel(
    _edge_body,
    out_type=[jax.ShapeDtypeStruct((N_UP, HH), jnp.float32),
              jax.ShapeDtypeStruct((N_UP, HH), jnp.float32)],
    mesh=plsc.VectorSubcoreMesh(core_axis_name="c", subcore_axis_name="s"),
    compiler_params=pltpu.CompilerParams(use_tc_tiling_on_sc=False),
    scratch_types=[
        pltpu.VMEM((4 * KB, 2, STREAM), jnp.int32),
        pltpu.VMEM((2 * KB, STREAM, HH), jnp.float32),
        pltpu.VMEM((ZR, HH), jnp.float32),
        pltpu.VMEM_SHARED((N_UP, HH), jnp.float32),
        pltpu.SemaphoreType.DMA((2,)),
        pltpu.SemaphoreType.DMA((2,)),
        pltpu.SemaphoreType.DMA,
    ],
)


def _pool_body(h_a, h_b, pidm, pooled_a, pooled_b, cnt,
               cid, rows, zbuf, ones_b, cbuf, accp, accc, gsem, ssem):
    c = lax.axis_index("c")
    s = lax.axis_index("s")
    _fill_zero_2d(zbuf, ZR)
    _fill_1d(ones_b, STREAM, 1.0)
    pltpu.sync_copy(zbuf.at[pl.ds(0, GP // NS)],
                    accp.at[pl.ds(s * (GP // NS), GP // NS)])

    @pl.when((c == 0) & (s == 0))
    def _():
        _fill_1d(cbuf, CNT, 0.0)
        pltpu.sync_copy(cbuf, accc)
    plsc.subcore_barrier()

    row0 = s * (NPT // STREAM)

    def pipeline(table, want_cnt):
        def blk(b, carry):
            rb = row0 + b * KB
            pltpu.sync_copy(pidm.at[pl.ds(rb, KB)], cid)
            gd = [pltpu.async_copy(table.at[cid.at[j, 0]], rows.at[j], gsem)
                  for j in range(KB)]
            for d in gd:
                d.wait()
            sd = [pltpu.async_copy(rows.at[j], accp.at[cid.at[j, 1]], ssem,
                                   add=True)
                  for j in range(KB)]
            if want_cnt:
                sd += [pltpu.async_copy(ones_b, accc.at[cid.at[j, 1]], ssem,
                                        add=True)
                       for j in range(KB)]
            for d in sd:
                d.wait()
            return carry
        lax.fori_loop(0, NBLK_P, blk, 0)

    @pl.when(c == 0)
    def _():
        pipeline(h_a, True)

    @pl.when(c == 1)
    def _():
        pipeline(h_b, False)
    plsc.subcore_barrier()

    @pl.when((c == 0) & (s == 0))
    def _():
        pltpu.sync_copy(accp, zbuf.at[pl.ds(0, GP)])
        pltpu.sync_copy(zbuf.at[pl.ds(0, GP)], pooled_a)
        pltpu.sync_copy(accc, cbuf)
        pltpu.sync_copy(cbuf, cnt)

    @pl.when((c == 1) & (s == 0))
    def _():
        pltpu.sync_copy(accp, zbuf.at[pl.ds(0, GP)])
        pltpu.sync_copy(zbuf.at[pl.ds(0, GP)], pooled_b)


_pool_call = pl.kernel(
    _pool_body,
    out_type=[jax.ShapeDtypeStruct((GP, HH), jnp.float32),
              jax.ShapeDtypeStruct((GP, HH), jnp.float32),
              jax.ShapeDtypeStruct((CNT,), jnp.float32)],
    mesh=plsc.VectorSubcoreMesh(core_axis_name="c", subcore_axis_name="s"),
    compiler_params=pltpu.CompilerParams(use_tc_tiling_on_sc=False),
    scratch_types=[
        pltpu.VMEM((KB, 2, STREAM), jnp.int32),
        pltpu.VMEM((KB, STREAM, HH), jnp.float32),
        pltpu.VMEM((ZR, HH), jnp.float32),
        pltpu.VMEM((STREAM,), jnp.float32),
        pltpu.VMEM((CNT,), jnp.float32),
        pltpu.VMEM_SHARED((GP, HH), jnp.float32),
        pltpu.VMEM_SHARED((CNT,), jnp.float32),
        pltpu.SemaphoreType.DMA,
        pltpu.SemaphoreType.DMA,
    ],
)


NP128 = N_UP * HH // 128
BS0 = 216
TBP = 696


def _tc0_body(xp, da, db, bde_a, bde_b, eb_a, eb_b, dinvp_o, hsa_o, hsb_o):
    deg = da[:] + db[:] + 1.0
    dvs = lax.rsqrt(deg)
    dp = jnp.broadcast_to(dvs.reshape(BS0, 2, 8, 1),
                          (BS0, 2, 8, 16)).reshape(BS0 * 2, 128)
    h0a = jnp.dot(xp[:], bde_a[:], preferred_element_type=jnp.float32) + eb_a[:]
    h0b = jnp.dot(xp[:], bde_b[:], preferred_element_type=jnp.float32) + eb_b[:]
    dinvp_o[:] = dp
    hsa_o[:] = h0a * dp
    hsb_o[:] = h0b * dp


def _tc0(xp, da, db, bde_a, bde_b, eb_a, eb_b):
    return pl.pallas_call(
        _tc0_body,
        grid=(N_UP // HH // BS0,),
        in_specs=[
            pl.BlockSpec((2 * BS0, 2 * 8), lambda b: (b, 0)),
            pl.BlockSpec((BS0, HH), lambda b: (b, 0)),
            pl.BlockSpec((BS0, HH), lambda b: (b, 0)),
            pl.BlockSpec((2 * 8, 128), lambda b: (0, 0)),
            pl.BlockSpec((2 * 8, 128), lambda b: (0, 0)),
            pl.BlockSpec((128,), lambda b: (0,)),
            pl.BlockSpec((128,), lambda b: (0,)),
        ],
        out_specs=[
            pl.BlockSpec((2 * BS0, 128), lambda b: (b, 0)),
            pl.BlockSpec((2 * BS0, 128), lambda b: (b, 0)),
            pl.BlockSpec((2 * BS0, 128), lambda b: (b, 0)),
        ],
        out_shape=[
            jax.ShapeDtypeStruct((NP128, 128), jnp.float32),
            jax.ShapeDtypeStruct((NP128, 128), jnp.float32),
            jax.ShapeDtypeStruct((NP128, 128), jnp.float32),
        ],
    )(xp, da, db, bde_a, bde_b, eb_a, eb_b)


def _layer_body(scale_out, aa, ab, ha, hb, dp, waa, wab, wba, wbb,
                ba, bb2, oa, ob):
    dv = dp[:]
    ta = (aa[:] + ha[:]) * dv
    tb = (ab[:] + hb[:]) * dv
    hra = jnp.dot(ta, waa[:], preferred_element_type=jnp.float32)
    hra = hra + jnp.dot(tb, wba[:], preferred_element_type=jnp.float32)
    hra = jnp.maximum(hra + ba[:], 0.0)
    hrb = jnp.dot(ta, wab[:], preferred_element_type=jnp.float32)
    hrb = hrb + jnp.dot(tb, wbb[:], preferred_element_type=jnp.float32)
    hrb = jnp.maximum(hrb + bb2[:], 0.0)
    if scale_out:
        hra = hra * dv
        hrb = hrb * dv
    oa[:] = hra
    ob[:] = hrb


def _tc_layer(agg_a, agg_b, hs_a, hs_b, dinv_p, waa, wab, wba, wbb,
              ba, bb2, scale_out):
    blk = pl.BlockSpec((TBP, 128), lambda b: (b, 0))
    wblk = pl.BlockSpec((128, 128), lambda b: (0, 0))
    return pl.pallas_call(
        functools.partial(_layer_body, scale_out),
        grid=(NP128 // TBP,),
        in_specs=[blk, blk, blk, blk, blk, wblk, wblk, wblk, wblk,
                  pl.BlockSpec((128,), lambda b: (0,)),
                  pl.BlockSpec((128,), lambda b: (0,))],
        out_specs=[blk, blk],
        out_shape=[
            jax.ShapeDtypeStruct((NP128, 128), jnp.float32),
            jax.ShapeDtypeStruct((NP128, 128), jnp.float32),
        ],
    )(agg_a, agg_b, hs_a, hs_b, dinv_p, waa, wab, wba, wbb, ba, bb2)


def _final_body(pa, pb, cn, fw, fb, out):
    p = jnp.concatenate([pa[:G, :], pb[:G, :]], axis=1)
    mean = p / jnp.maximum(cn[:G], 1.0)[:, None]
    out[:] = jnp.dot(mean, fw[:], preferred_element_type=jnp.float32) + fb[:]


def _tc_final(pooled_a, pooled_b, cnt, fc_W, fc_b):
    return pl.pallas_call(
        _final_body,
        out_shape=jax.ShapeDtypeStruct((G, OUT_F), jnp.float32),
    )(pooled_a, pooled_b, cnt, fc_W, fc_b)


def kernel(x, edge_index, batch, embed_W, embed_b, conv_W, conv_b, fc_W, fc_b):
    src = edge_index[0]
    dst = edge_index[1]
    pad = EPAD - E
    srcm = jnp.concatenate(
        [src, jnp.zeros((pad,), jnp.int32)]).reshape(EROWS, 1, STREAM)
    dstm = jnp.concatenate(
        [dst, jnp.full((pad,), N, jnp.int32)]).reshape(EROWS, 1, STREAM)
    edg = jnp.concatenate([srcm, dstm], axis=1)

    nidm = jnp.concatenate(
        [jnp.arange(N, dtype=jnp.int32),
         jnp.zeros((NODE_PAD - N,), jnp.int32)]).reshape(-1, 1, STREAM)
    batm = jnp.concatenate(
        [batch, jnp.full((NODE_PAD - N,), G, jnp.int32)]).reshape(-1, 1, STREAM)
    pidm = jnp.concatenate([nidm, batm], axis=1)

    eye8 = jnp.eye(8, dtype=jnp.float32)
    bde_a = jnp.kron(eye8, embed_W[:, :HH])
    bde_b = jnp.kron(eye8, embed_W[:, HH:])
    eb_a = jnp.tile(embed_b[:HH], 8)
    eb_b = jnp.tile(embed_b[HH:], 8)

    deg_a, deg_b = _deg_call(edg)
    dinv_p, hs_a, hs_b = _tc0(x.reshape(N // 8, 16),
                              deg_a.reshape(N_UP // HH, HH),
                              deg_b.reshape(N_UP // HH, HH),
                              bde_a, bde_b, eb_a, eb_b)

    for i in range(L_LAYERS):
        wi = conv_W[i]
        waa = jnp.kron(eye8, wi[:HH, :HH])
        wab = jnp.kron(eye8, wi[:HH, HH:])
        wba = jnp.kron(eye8, wi[HH:, :HH])
        wbb = jnp.kron(eye8, wi[HH:, HH:])
        ba = jnp.tile(conv_b[i][:HH], 8)
        bb2 = jnp.tile(conv_b[i][HH:], 8)
        agg_a, agg_b = _edge_call(hs_a.reshape(N_UP, HH),
                                  hs_b.reshape(N_UP, HH), edg)
        hs_a, hs_b = _tc_layer(agg_a.reshape(NP128, 128),
                               agg_b.reshape(NP128, 128),
                               hs_a, hs_b, dinv_p, waa, wab, wba, wbb,
                               ba, bb2, scale_out=(i < L_LAYERS - 1))

    pooled_a, pooled_b, cnt = _pool_call(hs_a.reshape(N_UP, HH),
                                         hs_b.reshape(N_UP, HH), pidm)
    return _tc_final(pooled_a, pooled_b, cnt, fc_W, fc_b)
